# baseline (speedup 1.0000x reference)
import functools

import jax
import jax.numpy as jnp
from jax import lax
from jax.experimental import pallas as pl
from jax.experimental.pallas import tpu as pltpu
from jax.experimental.pallas import tpu_sc as plsc

_BATCH = 16384
_ED = 32
_LANES = 128

_NC, _NS = 2, 16
_NW = _NC * _NS
_BPW = _BATCH // _NW
_HALF = _BPW // 2
_CHUNK = 128

_MLP_BLOCK = 4096


def _gather2(ia, ib, table):
    mesh = plsc.VectorSubcoreMesh(core_axis_name="c", subcore_axis_name="s")
    out = jax.ShapeDtypeStruct((_BATCH, _LANES), jnp.float32)
    fp = jnp.float32

    @functools.partial(
        pl.kernel, out_type=(out, out), mesh=mesh,
        scratch_types=[
            pltpu.VMEM((_BPW,), jnp.int32), pltpu.VMEM((_BPW,), jnp.int32),
            pltpu.VMEM((_HALF, _LANES), fp), pltpu.VMEM((_HALF, _LANES), fp),
            pltpu.SemaphoreType.DMA, pltpu.SemaphoreType.DMA,
            pltpu.SemaphoreType.DMA, pltpu.SemaphoreType.DMA,
        ])
    def gather_kernel(ia_hbm, ib_hbm, t_hbm, oa_hbm, ob_hbm,
                      iva, ivb, rows0, rows1, sg0, sg1, sw0, sw1):
        wid = lax.axis_index("s") * _NC + lax.axis_index("c")
        base = wid * _BPW
        i_hbms = (ia_hbm, ib_hbm)
        o_hbms = (oa_hbm, ob_hbm)
        ivs = (iva, ivb)
        rows = (rows0, rows1)
        sgs = (sg0, sg1)
        sws = (sw0, sw1)
        for k in range(2):
            pltpu.sync_copy(i_hbms[k].at[pl.ds(base, _BPW)], ivs[k])
        wdescs = []
        items = [(k, h) for k in range(2) for h in range(2)]
        for i, (k, h) in enumerate(items):
            b = i % 2
            if i >= 2:
                wdescs[i - 2].wait()
            gd = []
            for c in range(_HALF // _CHUNK):
                isl = pl.ds(h * _HALF + c * _CHUNK, _CHUNK)
                gd.append(pltpu.async_copy(
                    t_hbm.at[ivs[k].at[isl]],
                    rows[b].at[pl.ds(c * _CHUNK, _CHUNK)], sgs[b]))
            for d in gd:
                d.wait()
            wdescs.append(pltpu.async_copy(
                rows[b], o_hbms[k].at[pl.ds(base + h * _HALF, _HALF)], sws[b]))
        wdescs[-2].wait()
        wdescs[-1].wait()

    return gather_kernel(ia, ib, table)


def _mlp_partial_body(e1_ref, e2_ref, w1_ref, b1_ref, h_ref):
    h = b1_ref[...]
    h = h + jnp.dot(e1_ref[:, 0:_ED], w1_ref[_ED:2 * _ED, :],
                    preferred_element_type=jnp.float32)
    h = h + jnp.dot(e2_ref[:, _ED:2 * _ED], w1_ref[2 * _ED:3 * _ED, :],
                    preferred_element_type=jnp.float32)
    h_ref[...] = h


def _mlp_final_body(h_ref, e0_ref, e3_ref, w1_ref, w2_ref, b2_ref, o_ref):
    h = h_ref[...]
    h = h + jnp.dot(e0_ref[:, 0:_ED], w1_ref[0:_ED, :],
                    preferred_element_type=jnp.float32)
    h = h + jnp.dot(e3_ref[:, _ED:2 * _ED], w1_ref[3 * _ED:4 * _ED, :],
                    preferred_element_type=jnp.float32)
    h = jnp.maximum(h, 0.0)
    o_ref[...] = jnp.dot(h, w2_ref[...],
                         preferred_element_type=jnp.float32) + b2_ref[...]


def _mlp(es1, es2, eb0, eb3, W1, b1, W2, b2):
    full = lambda i: (0, 0)
    espec = lambda: pl.BlockSpec((_MLP_BLOCK, _LANES), lambda i: (i, 0))
    hspec = pl.BlockSpec((_MLP_BLOCK, 64), lambda i: (i, 0))
    grid = (_BATCH // _MLP_BLOCK,)
    h0 = pl.pallas_call(
        _mlp_partial_body,
        grid=grid,
        in_specs=[espec(), espec(),
                  pl.BlockSpec((128, 64), full),
                  pl.BlockSpec((1, 64), full)],
        out_specs=hspec,
        out_shape=jax.ShapeDtypeStruct((_BATCH, 64), jnp.float32),
    )(es1, es2, W1, b1.reshape(1, 64))
    return pl.pallas_call(
        _mlp_final_body,
        grid=grid,
        in_specs=[hspec, espec(), espec(),
                  pl.BlockSpec((128, 64), full),
                  pl.BlockSpec((64, 32), full),
                  pl.BlockSpec((1, 32), full)],
        out_specs=pl.BlockSpec((_MLP_BLOCK, 32), lambda i: (i, 0)),
        out_shape=jax.ShapeDtypeStruct((_BATCH, 32), jnp.float32),
    )(h0, eb0, eb3, W1, W2, b2.reshape(1, 32))


def kernel(room_id, hotel, room_type, room_name,
           room_table, hotel_table, room_type_table, room_name_table,
           W1, b1, W2, b2):
    def combo(ta, tb):
        z = jnp.zeros((ta.shape[0], _LANES), jnp.float32)
        return z.at[:, 0:_ED].set(ta).at[:, _ED:2 * _ED].set(tb)

    es1, es2 = _gather2(hotel, room_type, combo(hotel_table, room_type_table))
    eb0, eb3 = _gather2(room_id, room_name, combo(room_table, room_name_table))
    return _mlp(es1, es2, eb0, eb3, W1, b1, W2, b2)


# speedup vs baseline: 3.1572x; 3.1572x over previous
import functools

import jax
import jax.numpy as jnp
from jax import lax
from jax.experimental import pallas as pl
from jax.experimental.pallas import tpu as pltpu
from jax.experimental.pallas import tpu_sc as plsc

_BATCH = 16384
_ED = 32
_LANES = 128

_NC, _NS = 2, 16
_NW = _NC * _NS
_BPW = _BATCH // _NW
_HALF = _BPW // 2
_CHUNK = 128

_MLP_BLOCK = 4096


def _gather2(ia, ib, table):
    mesh = plsc.VectorSubcoreMesh(core_axis_name="c", subcore_axis_name="s")
    out = jax.ShapeDtypeStruct((_BATCH, _LANES), jnp.float32)
    fp = jnp.float32

    @functools.partial(
        pl.kernel, out_type=(out, out), mesh=mesh,
        scratch_types=[
            pltpu.VMEM((_BPW,), jnp.int32), pltpu.VMEM((_BPW,), jnp.int32),
            pltpu.VMEM((_HALF, _LANES), fp), pltpu.VMEM((_HALF, _LANES), fp),
            pltpu.SemaphoreType.DMA, pltpu.SemaphoreType.DMA,
            pltpu.SemaphoreType.DMA, pltpu.SemaphoreType.DMA,
        ])
    def gather_kernel(ia_hbm, ib_hbm, t_hbm, oa_hbm, ob_hbm,
                      iva, ivb, rows0, rows1, sg0, sg1, sw0, sw1):
        wid = lax.axis_index("s") * _NC + lax.axis_index("c")
        base = wid * _BPW
        i_hbms = (ia_hbm, ib_hbm)
        o_hbms = (oa_hbm, ob_hbm)
        ivs = (iva, ivb)
        rows = (rows0, rows1)
        sgs = (sg0, sg1)
        sws = (sw0, sw1)
        for k in range(2):
            pltpu.sync_copy(i_hbms[k].at[pl.ds(base, _BPW)], ivs[k])
        wdescs = []
        items = [(k, h) for k in range(2) for h in range(2)]
        for i, (k, h) in enumerate(items):
            b = i % 2
            if i >= 2:
                wdescs[i - 2].wait()
            gd = []
            for c in range(_HALF // _CHUNK):
                isl = pl.ds(h * _HALF + c * _CHUNK, _CHUNK)
                gd.append(pltpu.async_copy(
                    t_hbm.at[ivs[k].at[isl]],
                    rows[b].at[pl.ds(c * _CHUNK, _CHUNK)], sgs[b]))
            for d in gd:
                d.wait()
            wdescs.append(pltpu.async_copy(
                rows[b], o_hbms[k].at[pl.ds(base + h * _HALF, _HALF)], sws[b]))
        wdescs[-2].wait()
        wdescs[-1].wait()

    return gather_kernel(ia, ib, table)


def _mlp_partial_body(e1_ref, e2_ref, w1_ref, b1_ref, h_ref):
    h = b1_ref[...]
    h = h + jnp.dot(e1_ref[:, 0:_ED], w1_ref[_ED:2 * _ED, :],
                    preferred_element_type=jnp.float32)
    h = h + jnp.dot(e2_ref[:, _ED:2 * _ED], w1_ref[2 * _ED:3 * _ED, :],
                    preferred_element_type=jnp.float32)
    h_ref[...] = h


def _mlp_final_body(h_ref, e0_ref, e3_ref, w1_ref, w2_ref, b2_ref, o_ref):
    h = h_ref[...]
    h = h + jnp.dot(e0_ref[:, 0:_ED], w1_ref[0:_ED, :],
                    preferred_element_type=jnp.float32)
    h = h + jnp.dot(e3_ref[:, _ED:2 * _ED], w1_ref[3 * _ED:4 * _ED, :],
                    preferred_element_type=jnp.float32)
    h = jnp.maximum(h, 0.0)
    o_ref[...] = jnp.dot(h, w2_ref[...],
                         preferred_element_type=jnp.float32) + b2_ref[...]


def _mlp(es1, es2, eb0, eb3, W1, b1, W2, b2):
    full = lambda i: (0, 0)
    espec = lambda: pl.BlockSpec((_MLP_BLOCK, _LANES), lambda i: (i, 0))
    hspec = pl.BlockSpec((_MLP_BLOCK, 64), lambda i: (i, 0))
    grid = (_BATCH // _MLP_BLOCK,)
    h0 = pl.pallas_call(
        _mlp_partial_body,
        grid=grid,
        in_specs=[espec(), espec(),
                  pl.BlockSpec((128, 64), full),
                  pl.BlockSpec((1, 64), full)],
        out_specs=hspec,
        out_shape=jax.ShapeDtypeStruct((_BATCH, 64), jnp.float32),
    )(es1, es2, W1, b1.reshape(1, 64))
    return pl.pallas_call(
        _mlp_final_body,
        grid=grid,
        in_specs=[hspec, espec(), espec(),
                  pl.BlockSpec((128, 64), full),
                  pl.BlockSpec((64, 32), full),
                  pl.BlockSpec((1, 32), full)],
        out_specs=pl.BlockSpec((_MLP_BLOCK, 32), lambda i: (i, 0)),
        out_shape=jax.ShapeDtypeStruct((_BATCH, 32), jnp.float32),
    )(h0, eb0, eb3, W1, W2, b2.reshape(1, 32))


def kernel(room_id, hotel, room_type, room_name,
           room_table, hotel_table, room_type_table, room_name_table,
           W1, b1, W2, b2):
    def combo(ta, tb):
        z = jnp.zeros((ta.shape[0], _LANES - 2 * _ED), jnp.float32)
        return jnp.concatenate([ta, tb, z], axis=1)

    es1, es2 = _gather2(hotel, room_type, combo(hotel_table, room_type_table))
    eb0, eb3 = _gather2(room_id, room_name, combo(room_table, room_name_table))
    return _mlp(es1, es2, eb0, eb3, W1, b1, W2, b2)
